# baseline (device time: 9101 ns/iter reference)
import jax
import jax.numpy as jnp
from jax import lax
from jax.experimental import pallas as pl
from jax.experimental.pallas import tpu as pltpu

N_GLOBAL = 1024
EPS = 1e-5


def kernel(x, gamma, beta):
    m, n = x.shape

    def body(x_ref, g_ref, b_ref, out_ref, stats_ref, recv_ref, send_sem, recv_sem):
        my_x = lax.axis_index("x")
        my_y = lax.axis_index("y")
        nbr = (my_x, 1 - my_y)

        barrier_sem = pltpu.get_barrier_semaphore()
        pl.semaphore_signal(
            barrier_sem, inc=1, device_id=nbr, device_id_type=pl.DeviceIdType.MESH
        )
        pl.semaphore_wait(barrier_sem, 1)

        xv = x_ref[:, :]
        s2 = jnp.sum(xv, axis=1, keepdims=True)
        q2 = jnp.sum(xv * xv, axis=1, keepdims=True)
        stats_ref[:, :] = jnp.zeros((16, 128), jnp.float32)

        rdma = pltpu.make_async_remote_copy(
            src_ref=stats_ref,
            dst_ref=recv_ref,
            send_sem=send_sem,
            recv_sem=recv_sem,
            device_id=nbr,
            device_id_type=pl.DeviceIdType.MESH,
        )
        rdma.start()
        rdma.wait()

        total_sum = s2 * 2.0 + recv_ref[0, 0]
        total_sq = q2 * 2.0
        mean = total_sum / N_GLOBAL
        var = total_sq / N_GLOBAL - mean * mean
        inv = lax.rsqrt(var + EPS)
        out_ref[:, :] = g_ref[0:1, :] * ((xv - mean) * inv) + b_ref[0:1, :]

    return pl.pallas_call(
        body,
        out_shape=jax.ShapeDtypeStruct((m, n), x.dtype),
        in_specs=[
            pl.BlockSpec(memory_space=pltpu.VMEM),
            pl.BlockSpec(memory_space=pltpu.VMEM),
            pl.BlockSpec(memory_space=pltpu.VMEM),
        ],
        out_specs=pl.BlockSpec(memory_space=pltpu.VMEM),
        scratch_shapes=[
            pltpu.VMEM((16, 128), jnp.float32),
            pltpu.VMEM((16, 128), jnp.float32),
            pltpu.SemaphoreType.DMA,
            pltpu.SemaphoreType.DMA,
        ],
        compiler_params=pltpu.CompilerParams(collective_id=0),
    )(x, gamma.reshape(1, n), beta.reshape(1, n))


# device time: 8831 ns/iter; 1.0306x vs baseline; 1.0306x over previous
import jax
import jax.numpy as jnp
from jax import lax
from jax.experimental import pallas as pl
from jax.experimental.pallas import tpu as pltpu

N_GLOBAL = 1024
EPS = 1e-5


def kernel(x, gamma, beta):
    m, n = x.shape

    def body(x_ref, g_ref, b_ref, out_ref, stats_ref, recv_ref, send_sem, recv_sem):
        my_x = lax.axis_index("x")
        my_y = lax.axis_index("y")
        nbr = (my_x, 1 - my_y)

        barrier_sem = pltpu.get_barrier_semaphore()
        pl.semaphore_signal(
            barrier_sem, inc=1, device_id=nbr, device_id_type=pl.DeviceIdType.MESH
        )
        pl.semaphore_wait(barrier_sem, 1)

        xv = x_ref[:, :]
        s = jnp.sum(xv, axis=1)
        q = jnp.sum(xv * xv, axis=1)
        stats_ref[0:8, :] = s.reshape(8, 128)
        stats_ref[8:16, :] = q.reshape(8, 128)

        rdma = pltpu.make_async_remote_copy(
            src_ref=stats_ref,
            dst_ref=recv_ref,
            send_sem=send_sem,
            recv_sem=recv_sem,
            device_id=nbr,
            device_id_type=pl.DeviceIdType.MESH,
        )
        rdma.start()
        rdma.wait()

        tot = stats_ref[:, :] + recv_ref[:, :]

        lane = lax.broadcasted_iota(jnp.int32, (128, 128), 1)
        sub = lax.broadcasted_iota(jnp.int32, (128, 128), 0)
        diag = lane == sub
        for a in range(8):
            srow = tot[a : a + 1, :]
            qrow = tot[8 + a : 9 + a, :]
            scol = jnp.sum(
                jnp.where(diag, jnp.broadcast_to(srow, (128, 128)), 0.0),
                axis=1,
                keepdims=True,
            )
            qcol = jnp.sum(
                jnp.where(diag, jnp.broadcast_to(qrow, (128, 128)), 0.0),
                axis=1,
                keepdims=True,
            )
            mean = scol / N_GLOBAL
            var = qcol / N_GLOBAL - mean * mean
            inv = lax.rsqrt(var + EPS)
            lo, hi = a * 128, (a + 1) * 128
            xa = xv[lo:hi, :]
            out_ref[lo:hi, :] = g_ref[0:1, :] * ((xa - mean) * inv) + b_ref[0:1, :]

    return pl.pallas_call(
        body,
        out_shape=jax.ShapeDtypeStruct((m, n), x.dtype),
        in_specs=[
            pl.BlockSpec(memory_space=pltpu.VMEM),
            pl.BlockSpec(memory_space=pltpu.VMEM),
            pl.BlockSpec(memory_space=pltpu.VMEM),
        ],
        out_specs=pl.BlockSpec(memory_space=pltpu.VMEM),
        scratch_shapes=[
            pltpu.VMEM((16, 128), jnp.float32),
            pltpu.VMEM((16, 128), jnp.float32),
            pltpu.SemaphoreType.DMA,
            pltpu.SemaphoreType.DMA,
        ],
        compiler_params=pltpu.CompilerParams(collective_id=0),
    )(x, gamma.reshape(1, n), beta.reshape(1, n))
